# baseline (speedup 1.0000x reference)
import functools

import jax
import jax.numpy as jnp
from jax import lax
from jax.experimental import pallas as pl
from jax.experimental.pallas import tpu as pltpu
from jax.experimental.pallas import tpu_sc as plsc

D_TOKEN = 16
NCOLS = 26
BATCH = 16384
FLAT = BATCH * NCOLS
NUM_CORES = 2
NUM_SUBCORES = 16
NW = NUM_CORES * NUM_SUBCORES
PER_W = FLAT // NW
ROWS_W = PER_W // 128
OFF_PERIOD = 13


def _make_sc_gather():
    mesh = plsc.VectorSubcoreMesh(core_axis_name="c", subcore_axis_name="s")

    @functools.partial(
        pl.kernel,
        mesh=mesh,
        compiler_params=pltpu.CompilerParams(use_tc_tiling_on_sc=False),
        out_type=jax.ShapeDtypeStruct((FLAT, D_TOKEN), jnp.float32),
        scratch_types=[
            pltpu.VMEM((ROWS_W, 128), jnp.int32),
            pltpu.VMEM((OFF_PERIOD, 128), jnp.int32),
            pltpu.VMEM((128, D_TOKEN), jnp.float32),
            pltpu.SemaphoreType.DMA,
        ],
    )
    def sc_gather(x_hbm, offs_hbm, table_hbm, out_hbm, idx_v, offs_v, rows_v, sem):
        wid = lax.axis_index("s") * NUM_CORES + lax.axis_index("c")
        rbase = wid * ROWS_W

        pltpu.sync_copy(x_hbm.at[pl.ds(rbase, ROWS_W)], idx_v)
        pltpu.sync_copy(offs_hbm, offs_v)

        def add_body(j, carry):
            jm = lax.rem(j, OFF_PERIOD)
            for k in range(128 // 16):
                sl = pl.ds(k * 16, 16)
                idx_v[j, sl] = idx_v[j, sl] + offs_v[jm, sl]
            return carry

        lax.fori_loop(0, ROWS_W, add_body, 0)

        def gather_body(j, carry):
            pltpu.async_copy(table_hbm.at[idx_v.at[j]], rows_v, sem).wait()
            pltpu.sync_copy(rows_v, out_hbm.at[pl.ds((rbase + j) * 128, 128)])
            return carry

        lax.fori_loop(0, ROWS_W, gather_body, 0)

    return sc_gather


_sc_gather = _make_sc_gather()


@jax.jit
def kernel(x_cat, table, category_offsets):
    B, C = x_cat.shape
    x_flat = x_cat.reshape(FLAT // 128, 128)
    offs_pat = jnp.tile(category_offsets, (128 * OFF_PERIOD) // NCOLS).reshape(
        OFF_PERIOD, 128
    )
    out_flat = _sc_gather(x_flat, offs_pat, table)
    return out_flat.reshape(B, C, D_TOKEN)


# speedup vs baseline: 1.3665x; 1.3665x over previous
import functools

import jax
import jax.numpy as jnp
from jax import lax
from jax.experimental import pallas as pl
from jax.experimental.pallas import tpu as pltpu
from jax.experimental.pallas import tpu_sc as plsc

D_TOKEN = 16
NCOLS = 26
BATCH = 16384
NBLK = BATCH // 128
UNITS = NCOLS * NBLK
NUM_CORES = 2
NUM_SUBCORES = 16
NW = NUM_CORES * NUM_SUBCORES
UNITS_W = UNITS // NW


def _make_sc_gather():
    mesh = plsc.VectorSubcoreMesh(core_axis_name="c", subcore_axis_name="s")

    @functools.partial(
        pl.kernel,
        mesh=mesh,
        compiler_params=pltpu.CompilerParams(
            use_tc_tiling_on_sc=False, needs_layout_passes=False
        ),
        out_type=jax.ShapeDtypeStruct((NCOLS, 2, NBLK, 8, 128), jnp.float32),
        scratch_types=[
            pltpu.VMEM((UNITS_W, 128), jnp.int32),
            pltpu.VMEM((32,), jnp.int32),
            pltpu.VMEM((128, D_TOKEN), jnp.float32),
            pltpu.VMEM((128, D_TOKEN), jnp.float32),
            pltpu.VMEM((D_TOKEN, 128), jnp.float32),
            pltpu.VMEM((D_TOKEN, 128), jnp.float32),
            pltpu.SemaphoreType.DMA,
            pltpu.SemaphoreType.DMA,
        ],
    )
    def sc_gather(
        x_hbm, offs_hbm, table_hbm, out_hbm,
        idx_v, offs_v, rows0, rows1, tr0, tr1, sem_g, sem_o,
    ):
        wid = lax.axis_index("s") * NUM_CORES + lax.axis_index("c")
        ubase = wid * UNITS_W

        pltpu.sync_copy(x_hbm.at[pl.ds(ubase, UNITS_W)], idx_v)
        pltpu.sync_copy(offs_hbm, offs_v)

        def add_body(j, carry):
            col = lax.div(ubase + j, NBLK)
            off16 = plsc.load_gather(offs_v, [jnp.full((16,), 0, jnp.int32) + col])
            for k in range(128 // 16):
                sl = pl.ds(k * 16, 16)
                idx_v[j, sl] = idx_v[j, sl] + off16
            return carry

        lax.fori_loop(0, UNITS_W, add_body, 0)

        rows_bufs = (rows0, rows1)
        tr_bufs = (tr0, tr1)
        iota = lax.iota(jnp.int32, 16)

        def gather_start(j, p):
            pltpu.async_copy(table_hbm.at[idx_v.at[j]], rows_bufs[p], sem_g)

        gather_start(0, 0)

        def unit_body(j, carry):
            u = ubase + j
            col = lax.div(u, NBLK)
            blk = lax.rem(u, NBLK)
            for p in range(2):

                @pl.when(lax.rem(j, 2) == p)
                def _():
                    rows_v = rows_bufs[p]
                    tr_v = tr_bufs[p]

                    @pl.when(j + 1 < UNITS_W)
                    def _():
                        gather_start(j + 1, 1 - p)

                    pltpu.make_async_copy(
                        table_hbm.at[pl.ds(0, 128)], rows_v, sem_g
                    ).wait()

                    for dm in range(D_TOKEN):
                        dmv = jnp.full((16,), dm, jnp.int32)
                        for k in range(8):
                            vals = plsc.load_gather(
                                rows_v, [iota + k * 16, dmv]
                            )
                            tr_v[dm, pl.ds(k * 16, 16)] = vals

                    @pl.when(j >= 2)
                    def _():
                        pltpu.make_async_copy(
                            x_hbm.at[pl.ds(0, 16)], tr_v, sem_o
                        ).wait()

                    pltpu.async_copy(
                        tr_v.at[pl.ds(0, 8)], out_hbm.at[col, 0, blk], sem_o
                    )
                    pltpu.async_copy(
                        tr_v.at[pl.ds(8, 8)], out_hbm.at[col, 1, blk], sem_o
                    )

            return carry

        lax.fori_loop(0, UNITS_W, unit_body, 0)

        for p in range(2):
            pltpu.make_async_copy(x_hbm.at[pl.ds(0, 16)], tr_bufs[p], sem_o).wait()

    return sc_gather


_sc_gather = _make_sc_gather()


@jax.jit
def kernel(x_cat, table, category_offsets):
    B, C = x_cat.shape
    xt2d = x_cat.T.reshape(UNITS, 128)
    offs_pad = jnp.pad(category_offsets, (0, 32 - NCOLS))
    out5 = _sc_gather(xt2d, offs_pad, table)
    return out5.transpose(2, 4, 0, 1, 3).reshape(B, C, D_TOKEN)


# speedup vs baseline: 1.4924x; 1.0921x over previous
import functools

import jax
import jax.numpy as jnp
from jax import lax
from jax.experimental import pallas as pl
from jax.experimental.pallas import tpu as pltpu
from jax.experimental.pallas import tpu_sc as plsc

D_TOKEN = 16
NCOLS = 26
BATCH = 16384
NBLK = BATCH // 128
UNITS = NCOLS * NBLK
NUM_CORES = 2
NUM_SUBCORES = 16
NW = NUM_CORES * NUM_SUBCORES
UNITS_W = UNITS // NW

VROWS = 2600000
LIN_ROWS = VROWS // 8
CH_LANES = 1024
N_CHUNKS = VROWS // CH_LANES
TAIL = VROWS - N_CHUNKS * CH_LANES
CPW = -(-N_CHUNKS // NW)


def _make_retile():
    mesh = plsc.VectorSubcoreMesh(core_axis_name="c", subcore_axis_name="s")

    @functools.partial(
        pl.kernel,
        mesh=mesh,
        compiler_params=pltpu.CompilerParams(needs_layout_passes=False),
        out_type=jax.ShapeDtypeStruct((LIN_ROWS, 128), jnp.float32),
        scratch_types=[
            pltpu.VMEM((D_TOKEN, CH_LANES), jnp.float32),
            pltpu.VMEM((D_TOKEN, CH_LANES), jnp.float32),
            pltpu.VMEM((CH_LANES // 8, 128), jnp.float32),
            pltpu.VMEM((CH_LANES // 8, 128), jnp.float32),
            pltpu.SemaphoreType.DMA,
            pltpu.SemaphoreType.DMA,
        ],
    )
    def retile(tt_hbm, tail_hbm, out_hbm, slab0, slab1, tr0, tr1, sem_r, sem_w):
        wid = lax.axis_index("s") * NUM_CORES + lax.axis_index("c")
        slabs = (slab0, slab1)
        trs = (tr0, tr1)
        iota = lax.iota(jnp.int32, 16)

        def read_start(ch, p):
            base = ch * CH_LANES
            pltpu.async_copy(
                tt_hbm.at[pl.ds(0, 8), pl.ds(base, CH_LANES)],
                slabs[p].at[pl.ds(0, 8)], sem_r,
            )
            pltpu.async_copy(
                tt_hbm.at[pl.ds(8, 8), pl.ds(base, CH_LANES)],
                slabs[p].at[pl.ds(8, 8)], sem_r,
            )

        @pl.when(wid < N_CHUNKS)
        def _():
            read_start(wid, 0)

        def chunk_body(m, carry):
            ch = wid + m * NW
            for p in range(2):

                @pl.when((lax.rem(m, 2) == p) & (ch < N_CHUNKS))
                def _():
                    slab = slabs[p]
                    tr = trs[p]

                    nxt = ch + NW

                    @pl.when(nxt < N_CHUNKS)
                    def _():
                        read_start(nxt, 1 - p)

                    pltpu.make_async_copy(
                        tt_hbm.at[pl.ds(0, 8), pl.ds(0, CH_LANES)],
                        slab.at[pl.ds(0, 8)], sem_r,
                    ).wait()
                    pltpu.make_async_copy(
                        tt_hbm.at[pl.ds(0, 8), pl.ds(0, CH_LANES)],
                        slab.at[pl.ds(8, 8)], sem_r,
                    ).wait()

                    @pl.when(m >= 2)
                    def _():
                        pltpu.make_async_copy(
                            out_hbm.at[pl.ds(0, CH_LANES // 8)], tr, sem_w
                        ).wait()

                    def row_body(r, c2):
                        l0 = jnp.full((16,), 0, jnp.int32) + r * 8
                        for k in range(8):
                            vals = plsc.load_gather(slab, [iota, l0 + k])
                            tr[r, pl.ds(k * 16, 16)] = vals
                        return c2

                    lax.fori_loop(0, CH_LANES // 8, row_body, 0)

                    pltpu.async_copy(
                        tr, out_hbm.at[pl.ds(ch * (CH_LANES // 8), CH_LANES // 8)],
                        sem_w,
                    )

            return carry

        lax.fori_loop(0, CPW, chunk_body, 0)

        for p in range(2):
            pltpu.make_async_copy(
                out_hbm.at[pl.ds(0, CH_LANES // 8)], trs[p], sem_w
            ).wait()

        @pl.when(wid == 0)
        def _():
            pltpu.sync_copy(tail_hbm, tr0.at[pl.ds(0, TAIL // 8)])
            pltpu.sync_copy(
                tr0.at[pl.ds(0, TAIL // 8)],
                out_hbm.at[pl.ds(N_CHUNKS * (CH_LANES // 8), TAIL // 8)],
            )

    return retile


def _make_sc_gather():
    mesh = plsc.VectorSubcoreMesh(core_axis_name="c", subcore_axis_name="s")

    @functools.partial(
        pl.kernel,
        mesh=mesh,
        compiler_params=pltpu.CompilerParams(needs_layout_passes=False),
        out_type=jax.ShapeDtypeStruct((NCOLS, 2, NBLK, 8, 128), jnp.float32),
        scratch_types=[
            pltpu.VMEM((UNITS_W, 128), jnp.int32),
            pltpu.VMEM((UNITS_W, 128), jnp.int32),
            pltpu.VMEM((32,), jnp.int32),
            pltpu.VMEM((128, 128), jnp.float32),
            pltpu.VMEM((128, 128), jnp.float32),
            pltpu.VMEM((D_TOKEN, 128), jnp.float32),
            pltpu.VMEM((D_TOKEN, 128), jnp.float32),
            pltpu.SemaphoreType.DMA,
            pltpu.SemaphoreType.DMA,
        ],
    )
    def sc_gather(
        x_hbm, offs_hbm, tlin_hbm, out_hbm,
        gidx_v, sub_v, offs_v, grp0, grp1, tr0, tr1, sem_g, sem_o,
    ):
        wid = lax.axis_index("s") * NUM_CORES + lax.axis_index("c")
        ubase = wid * UNITS_W

        pltpu.sync_copy(x_hbm.at[pl.ds(ubase, UNITS_W)], gidx_v)
        pltpu.sync_copy(offs_hbm, offs_v)

        def add_body(j, carry):
            col = lax.div(ubase + j, NBLK)
            off16 = plsc.load_gather(offs_v, [jnp.full((16,), 0, jnp.int32) + col])
            for k in range(128 // 16):
                sl = pl.ds(k * 16, 16)
                idx = gidx_v[j, sl] + off16
                gidx_v[j, sl] = lax.shift_right_logical(idx, 3)
                sub_v[j, sl] = lax.bitwise_and(idx, 7) * 16
            return carry

        lax.fori_loop(0, UNITS_W, add_body, 0)

        grp_bufs = (grp0, grp1)
        tr_bufs = (tr0, tr1)
        iota = lax.iota(jnp.int32, 16)

        def gather_start(j, p):
            pltpu.async_copy(tlin_hbm.at[gidx_v.at[j]], grp_bufs[p], sem_g)

        gather_start(0, 0)

        def unit_body(j, carry):
            u = ubase + j
            col = lax.div(u, NBLK)
            blk = lax.rem(u, NBLK)
            for p in range(2):

                @pl.when(lax.rem(j, 2) == p)
                def _():
                    grp_v = grp_bufs[p]
                    tr_v = tr_bufs[p]

                    @pl.when(j + 1 < UNITS_W)
                    def _():
                        gather_start(j + 1, 1 - p)

                    pltpu.make_async_copy(
                        tlin_hbm.at[pl.ds(0, 128)], grp_v, sem_g
                    ).wait()

                    for k in range(8):
                        sl = pl.ds(k * 16, 16)
                        sub16 = sub_v[j, sl]
                        rows16 = iota + k * 16
                        for dm in range(D_TOKEN):
                            vals = plsc.load_gather(
                                grp_v, [rows16, sub16 + dm]
                            )
                            tr_v[dm, sl] = vals

                    @pl.when(j >= 2)
                    def _():
                        pltpu.make_async_copy(
                            x_hbm.at[pl.ds(0, 16)], tr_v, sem_o
                        ).wait()

                    pltpu.async_copy(
                        tr_v.at[pl.ds(0, 8)], out_hbm.at[col, 0, blk], sem_o
                    )
                    pltpu.async_copy(
                        tr_v.at[pl.ds(8, 8)], out_hbm.at[col, 1, blk], sem_o
                    )

            return carry

        lax.fori_loop(0, UNITS_W, unit_body, 0)

        for p in range(2):
            pltpu.make_async_copy(x_hbm.at[pl.ds(0, 16)], tr_bufs[p], sem_o).wait()

    return sc_gather


_retile = _make_retile()
_sc_gather = _make_sc_gather()


@jax.jit
def kernel(x_cat, table, category_offsets):
    B, C = x_cat.shape
    tail2d = table[VROWS - TAIL:].reshape(TAIL // 8, 128)
    tlin = _retile(table.T, tail2d)
    xt2d = x_cat.T.reshape(UNITS, 128)
    offs_pad = jnp.pad(category_offsets, (0, 32 - NCOLS))
    out5 = _sc_gather(xt2d, offs_pad, tlin)
    return out5.transpose(2, 4, 0, 1, 3).reshape(B, C, D_TOKEN)


# speedup vs baseline: 3.7575x; 2.5178x over previous
import functools

import jax
import jax.numpy as jnp
from jax import lax
from jax.experimental import pallas as pl
from jax.experimental.pallas import tpu as pltpu
from jax.experimental.pallas import tpu_sc as plsc

D_TOKEN = 16
NCOLS = 26
BATCH = 16384
NBLK = BATCH // 128
UNITS = NCOLS * NBLK
NUM_CORES = 2
NUM_SUBCORES = 16
NW = NUM_CORES * NUM_SUBCORES
UNITS_W = UNITS // NW

VROWS = 2600000
LIN_ROWS = VROWS // 8
CH_LANES = 1024
N_CHUNKS = VROWS // CH_LANES
TAIL = VROWS - N_CHUNKS * CH_LANES
CPW = -(-N_CHUNKS // NW)


def _make_retile():
    mesh = plsc.VectorSubcoreMesh(core_axis_name="c", subcore_axis_name="s")

    @functools.partial(
        pl.kernel,
        mesh=mesh,
        compiler_params=pltpu.CompilerParams(needs_layout_passes=False),
        out_type=jax.ShapeDtypeStruct((LIN_ROWS, 128), jnp.float32),
        scratch_types=[
            pltpu.VMEM((D_TOKEN, CH_LANES), jnp.float32),
            pltpu.VMEM((D_TOKEN, CH_LANES), jnp.float32),
            pltpu.VMEM((CH_LANES // 8, 128), jnp.float32),
            pltpu.VMEM((CH_LANES // 8, 128), jnp.float32),
            pltpu.SemaphoreType.DMA,
            pltpu.SemaphoreType.DMA,
        ],
    )
    def retile(tt_hbm, tail_hbm, out_hbm, slab0, slab1, tr0, tr1, sem_r, sem_w):
        wid = lax.axis_index("s") * NUM_CORES + lax.axis_index("c")
        slabs = (slab0, slab1)
        trs = (tr0, tr1)
        iota = lax.iota(jnp.int32, 16)

        def read_start(ch, p):
            base = ch * CH_LANES
            pltpu.async_copy(
                tt_hbm.at[pl.ds(0, 8), pl.ds(base, CH_LANES)],
                slabs[p].at[pl.ds(0, 8)], sem_r,
            )
            pltpu.async_copy(
                tt_hbm.at[pl.ds(8, 8), pl.ds(base, CH_LANES)],
                slabs[p].at[pl.ds(8, 8)], sem_r,
            )

        @pl.when(wid < N_CHUNKS)
        def _():
            read_start(wid, 0)

        def chunk_body(m, carry):
            ch = wid + m * NW
            for p in range(2):

                @pl.when((lax.rem(m, 2) == p) & (ch < N_CHUNKS))
                def _():
                    slab = slabs[p]
                    tr = trs[p]

                    nxt = ch + NW

                    @pl.when(nxt < N_CHUNKS)
                    def _():
                        read_start(nxt, 1 - p)

                    pltpu.make_async_copy(
                        tt_hbm.at[pl.ds(0, 8), pl.ds(0, CH_LANES)],
                        slab.at[pl.ds(0, 8)], sem_r,
                    ).wait()
                    pltpu.make_async_copy(
                        tt_hbm.at[pl.ds(0, 8), pl.ds(0, CH_LANES)],
                        slab.at[pl.ds(8, 8)], sem_r,
                    ).wait()

                    @pl.when(m >= 2)
                    def _():
                        pltpu.make_async_copy(
                            out_hbm.at[pl.ds(0, CH_LANES // 8)], tr, sem_w
                        ).wait()

                    c01 = lax.shift_right_logical(iota, 3)
                    i7_16 = lax.bitwise_and(iota, 7) * 16

                    def grp_body(k, c2):
                        idx0 = jnp.full((16,), 0, jnp.int32) + 2 * k + c01
                        for dm in range(D_TOKEN):
                            vals = slab[dm, pl.ds(k * 16, 16)]
                            plsc.store_scatter(tr, [idx0, i7_16 + dm], vals)
                        return c2

                    lax.fori_loop(0, CH_LANES // 16, grp_body, 0)

                    pltpu.async_copy(
                        tr, out_hbm.at[pl.ds(ch * (CH_LANES // 8), CH_LANES // 8)],
                        sem_w,
                    )

            return carry

        lax.fori_loop(0, CPW, chunk_body, 0)

        for p in range(2):
            pltpu.make_async_copy(
                out_hbm.at[pl.ds(0, CH_LANES // 8)], trs[p], sem_w
            ).wait()

        @pl.when(wid == 0)
        def _():
            pltpu.sync_copy(tail_hbm, tr0.at[pl.ds(0, TAIL // 8)])
            pltpu.sync_copy(
                tr0.at[pl.ds(0, TAIL // 8)],
                out_hbm.at[pl.ds(N_CHUNKS * (CH_LANES // 8), TAIL // 8)],
            )

    return retile


def _make_sc_gather():
    mesh = plsc.VectorSubcoreMesh(core_axis_name="c", subcore_axis_name="s")

    @functools.partial(
        pl.kernel,
        mesh=mesh,
        compiler_params=pltpu.CompilerParams(needs_layout_passes=False),
        out_type=jax.ShapeDtypeStruct((NCOLS, 2, NBLK, 8, 128), jnp.float32),
        scratch_types=[
            pltpu.VMEM((UNITS_W, 128), jnp.int32),
            pltpu.VMEM((UNITS_W, 128), jnp.int32),
            pltpu.VMEM((32,), jnp.int32),
            pltpu.VMEM((128, 128), jnp.float32),
            pltpu.VMEM((128, 128), jnp.float32),
            pltpu.VMEM((D_TOKEN, 128), jnp.float32),
            pltpu.VMEM((D_TOKEN, 128), jnp.float32),
            pltpu.SemaphoreType.DMA,
            pltpu.SemaphoreType.DMA,
        ],
    )
    def sc_gather(
        x_hbm, offs_hbm, tlin_hbm, out_hbm,
        gidx_v, sub_v, offs_v, grp0, grp1, tr0, tr1, sem_g, sem_o,
    ):
        wid = lax.axis_index("s") * NUM_CORES + lax.axis_index("c")
        ubase = wid * UNITS_W

        pltpu.sync_copy(x_hbm.at[pl.ds(ubase, UNITS_W)], gidx_v)
        pltpu.sync_copy(offs_hbm, offs_v)

        def add_body(j, carry):
            col = lax.div(ubase + j, NBLK)
            off16 = plsc.load_gather(offs_v, [jnp.full((16,), 0, jnp.int32) + col])
            for k in range(128 // 16):
                sl = pl.ds(k * 16, 16)
                idx = gidx_v[j, sl] + off16
                gidx_v[j, sl] = lax.shift_right_logical(idx, 3)
                sub_v[j, sl] = lax.bitwise_and(idx, 7) * 16
            return carry

        lax.fori_loop(0, UNITS_W, add_body, 0)

        grp_bufs = (grp0, grp1)
        tr_bufs = (tr0, tr1)
        iota = lax.iota(jnp.int32, 16)

        def gather_start(j, p):
            pltpu.async_copy(tlin_hbm.at[gidx_v.at[j]], grp_bufs[p], sem_g)

        gather_start(0, 0)

        def unit_body(j, carry):
            u = ubase + j
            col = lax.div(u, NBLK)
            blk = lax.rem(u, NBLK)
            for p in range(2):

                @pl.when(lax.rem(j, 2) == p)
                def _():
                    grp_v = grp_bufs[p]
                    tr_v = tr_bufs[p]

                    @pl.when(j + 1 < UNITS_W)
                    def _():
                        gather_start(j + 1, 1 - p)

                    pltpu.make_async_copy(
                        tlin_hbm.at[pl.ds(0, 128)], grp_v, sem_g
                    ).wait()

                    for k in range(8):
                        sl = pl.ds(k * 16, 16)
                        sub16 = sub_v[j, sl]
                        rows16 = iota + k * 16
                        for dm in range(D_TOKEN):
                            vals = plsc.load_gather(
                                grp_v, [rows16, sub16 + dm]
                            )
                            tr_v[dm, sl] = vals

                    @pl.when(j >= 2)
                    def _():
                        pltpu.make_async_copy(
                            x_hbm.at[pl.ds(0, 16)], tr_v, sem_o
                        ).wait()

                    pltpu.async_copy(
                        tr_v.at[pl.ds(0, 8)], out_hbm.at[col, 0, blk], sem_o
                    )
                    pltpu.async_copy(
                        tr_v.at[pl.ds(8, 8)], out_hbm.at[col, 1, blk], sem_o
                    )

            return carry

        lax.fori_loop(0, UNITS_W, unit_body, 0)

        for p in range(2):
            pltpu.make_async_copy(x_hbm.at[pl.ds(0, 16)], tr_bufs[p], sem_o).wait()

    return sc_gather


_retile = _make_retile()
_sc_gather = _make_sc_gather()


@jax.jit
def kernel(x_cat, table, category_offsets):
    B, C = x_cat.shape
    tail2d = table[VROWS - TAIL:].reshape(TAIL // 8, 128)
    tlin = _retile(table.T, tail2d)
    xt2d = x_cat.T.reshape(UNITS, 128)
    offs_pad = jnp.pad(category_offsets, (0, 32 - NCOLS))
    out5 = _sc_gather(xt2d, offs_pad, tlin)
    return out5.transpose(2, 4, 0, 1, 3).reshape(B, C, D_TOKEN)
